# baseline (speedup 1.0000x reference)
import functools
import jax
import jax.numpy as jnp
from jax import lax
from jax.experimental import pallas as pl
from jax.experimental.pallas import tpu as pltpu
from jax.experimental.pallas import tpu_sc as plsc

N = 10000
D = 128
H = D // 2
G = 64
E = 320000
NC = 2
NS = 16
NW = NC * NS
NP = 10240
RT = NP // NS
EP = 327680
CH = 80
C = 256
NB = 2
CPT = EP // NS // C


@functools.cache
def _mesh():
    return plsc.VectorSubcoreMesh(
        core_axis_name="c", subcore_axis_name="s",
        num_cores=NC, num_subcores=NS)


def _deg_body(dst_hbm, ones_hbm, zeros_hbm, out_hbm, dst_vm, ones_vm, zero_vm,
              deg_sh):
    c = lax.axis_index("c")
    s = lax.axis_index("s")
    r0 = s * RT
    pltpu.sync_copy(zeros_hbm, zero_vm)
    for k in range(RT // 128):
        pltpu.sync_copy(zero_vm, deg_sh.at[pl.ds(r0 + k * 128, 128)])
    pltpu.sync_copy(ones_hbm, ones_vm)
    pltpu.sync_copy(dst_hbm.at[pl.ds((s * NC + c) * CH, CH)], dst_vm)
    plsc.subcore_barrier()

    def body(j, carry):
        pltpu.sync_copy(ones_vm, deg_sh.at[dst_vm.at[j]], add=True)
        return carry

    lax.fori_loop(0, CH, body, 0)
    plsc.subcore_barrier()
    pltpu.sync_copy(deg_sh.at[pl.ds(r0, RT)],
                    out_hbm.at[pl.ds(c * NP + r0, RT)])


@functools.cache
def _deg_kernel():
    return pl.kernel(
        _deg_body,
        out_type=jax.ShapeDtypeStruct((NC * NP, 8), jnp.float32),
        mesh=_mesh(),
        scratch_types=[
            pltpu.VMEM((CH, 128), jnp.int32),
            pltpu.VMEM((128, 8), jnp.float32),
            pltpu.VMEM((128, 8), jnp.float32),
            pltpu.VMEM_SHARED((NP, 8), jnp.float32),
        ],
        compiler_params=pltpu.CompilerParams(use_tc_tiling_on_sc=False),
    )


def _agg_body(src_hbm, dst_hbm, hs_hbm, zeros_hbm, out_hbm, src_vm, dst_vm,
              rbs, acc, sgs, sss):
    c = lax.axis_index("c")
    s = lax.axis_index("s")
    r0 = s * RT
    pltpu.sync_copy(zeros_hbm, rbs[0].at[pl.ds(0, 128)])
    for k in range(RT // 128):
        pltpu.sync_copy(rbs[0].at[pl.ds(0, 128)],
                        acc.at[pl.ds(r0 + k * 128, 128)])
    pltpu.sync_copy(src_hbm.at[pl.ds(((c * NS + s) * CPT), CPT)], src_vm)
    pltpu.sync_copy(dst_hbm.at[pl.ds(s * CPT, CPT)], dst_vm)
    plsc.subcore_barrier()

    for b in range(NB):
        pltpu.async_copy(hs_hbm.at[src_vm.at[b]], rbs[b], sgs[b])

    def body(i, carry):
        j = NB * i
        for b in range(NB):
            pltpu.make_async_copy(hs_hbm.at[src_vm.at[j + b]], rbs[b],
                                  sgs[b]).wait()
            pltpu.async_copy(rbs[b], acc.at[dst_vm.at[j + b]], sss[b],
                             add=True)
        for b in range(NB):
            @pl.when(j + NB + b < CPT)
            def _(b=b):
                pltpu.make_async_copy(rbs[b], acc.at[dst_vm.at[0]],
                                      sss[b]).wait()
                pltpu.async_copy(hs_hbm.at[src_vm.at[j + NB + b]], rbs[b],
                                 sgs[b])
        return carry

    lax.fori_loop(0, CPT // NB, body, 0)
    for b in range(NB):
        pltpu.make_async_copy(rbs[b], acc.at[dst_vm.at[0]], sss[b]).wait()
    plsc.subcore_barrier()
    pltpu.sync_copy(acc.at[pl.ds(r0, RT)],
                    out_hbm.at[pl.ds(c * NP + r0, RT)])


@functools.cache
def _agg_kernel():
    return pl.kernel(
        _agg_body,
        out_type=jax.ShapeDtypeStruct((NC * NP, H), jnp.float32),
        mesh=_mesh(),
        scratch_types=[
            pltpu.VMEM((CPT, C), jnp.int32),
            pltpu.VMEM((CPT, C), jnp.int32),
            [pltpu.VMEM((C, H), jnp.float32) for _ in range(NB)],
            pltpu.VMEM_SHARED((NP, H), jnp.float32),
            [pltpu.SemaphoreType.DMA for _ in range(NB)],
            [pltpu.SemaphoreType.DMA for _ in range(NB)],
        ],
        compiler_params=pltpu.CompilerParams(use_tc_tiling_on_sc=False),
    )


def _tc1_body(x_ref, degp_ref, w_ref, hs_ref, dinv_ref):
    deg = degp_ref[0] + degp_ref[1] + 1.0
    dinv = lax.rsqrt(deg)
    dinv_ref[...] = dinv
    hs_ref[...] = jnp.dot(x_ref[...] * dinv, w_ref[...],
                          preferred_element_type=jnp.float32)


def _tc_mid_body(acc_ref, hsp_ref, dinv_ref, b_ref, w_ref, hs_ref):
    dinv = dinv_ref[...]
    hsp = hsp_ref[...]
    b = b_ref[...]
    w = w_ref[...]
    rid = lax.broadcasted_iota(jnp.int32, (NP, 1), 0)
    mask = rid < N
    tL = jnp.maximum((acc_ref[0] + hsp[:, :H]) * dinv + b[:, :H], 0.0)
    tR = jnp.maximum((acc_ref[1] + hsp[:, H:]) * dinv + b[:, H:], 0.0)
    tL = jnp.where(mask, tL, 0.0) * dinv
    tR = jnp.where(mask, tR, 0.0) * dinv
    hs_ref[...] = (jnp.dot(tL, w[:H, :], preferred_element_type=jnp.float32)
                   + jnp.dot(tR, w[H:, :], preferred_element_type=jnp.float32))


def _tc_fin_body(acc_ref, hsp_ref, dinv_ref, b_ref, batch_ref, wout_ref,
                 bout_ref, out_ref):
    dinv = dinv_ref[...]
    hsp = hsp_ref[...]
    b = b_ref[...]
    hL = jnp.maximum((acc_ref[0] + hsp[:, :H]) * dinv + b[:, :H], 0.0)
    hR = jnp.maximum((acc_ref[1] + hsp[:, H:]) * dinv + b[:, H:], 0.0)
    oh = (batch_ref[...] == lax.broadcasted_iota(jnp.int32, (NP, G), 1)
          ).astype(jnp.float32)
    dn = (((0,), (0,)), ((), ()))
    sumsL = lax.dot_general(oh, hL, dn, preferred_element_type=jnp.float32)
    sumsR = lax.dot_general(oh, hR, dn, preferred_element_type=jnp.float32)
    cnt = lax.dot_general(oh, jnp.ones((NP, 1), jnp.float32), dn,
                          preferred_element_type=jnp.float32)
    inv_cnt = 1.0 / jnp.maximum(cnt, 1.0)
    wout = wout_ref[...]
    out_ref[...] = (
        jnp.dot(sumsL * inv_cnt, wout[:H, :],
                preferred_element_type=jnp.float32)
        + jnp.dot(sumsR * inv_cnt, wout[H:, :],
                  preferred_element_type=jnp.float32)
        + bout_ref[...])


_tc1 = pl.pallas_call(
    _tc1_body,
    out_shape=[jax.ShapeDtypeStruct((NP, D), jnp.float32),
               jax.ShapeDtypeStruct((NP, 1), jnp.float32)],
)

_tc_mid = pl.pallas_call(
    _tc_mid_body,
    out_shape=jax.ShapeDtypeStruct((NP, D), jnp.float32),
)

_tc_fin = pl.pallas_call(
    _tc_fin_body,
    out_shape=jax.ShapeDtypeStruct((G, D), jnp.float32),
)


def kernel(x, edge_index, batch, W1, b1, W2, b2, W3, b3, Wout, bout):
    f32 = jnp.float32
    i32 = jnp.int32
    pad_e = jnp.full((EP - E,), N, i32)
    src = jnp.concatenate([edge_index[0].astype(i32), pad_e])
    dst = jnp.concatenate([edge_index[1].astype(i32), pad_e])
    src2 = src * 2
    src_w = jnp.stack([src2, src2 + 1]).reshape(NC * NS * CPT, C)
    dst_w = dst.reshape(NS * CPT, C)
    dst_deg = dst.reshape(NW * CH, 128)
    x_pad = jnp.pad(x.astype(f32), ((0, NP - N), (0, 0)))
    batch_pad = jnp.pad(batch.astype(i32), (0, NP - N),
                        constant_values=G).reshape(NP, 1)
    zeros8 = jnp.zeros((128, 8), f32)
    ones8 = jnp.ones((128, 8), f32)
    zerosH = jnp.zeros((128, H), f32)

    deg_out = _deg_kernel()(dst_deg, ones8, zeros8)
    degp = deg_out.reshape(NC, NP, 8)[:, :, :1]

    agg = _agg_kernel()
    hs1, dinv = _tc1(x_pad, degp, W1)
    acc1 = agg(src_w, dst_w, hs1.reshape(2 * NP, H), zerosH).reshape(
        NC, NP, H)
    hs2 = _tc_mid(acc1, hs1, dinv, b1.reshape(1, D), W2)
    acc2 = agg(src_w, dst_w, hs2.reshape(2 * NP, H), zerosH).reshape(
        NC, NP, H)
    hs3 = _tc_mid(acc2, hs2, dinv, b2.reshape(1, D), W3)
    acc3 = agg(src_w, dst_w, hs3.reshape(2 * NP, H), zerosH).reshape(
        NC, NP, H)
    out = _tc_fin(acc3, hs3, dinv, b3.reshape(1, D), batch_pad, Wout,
                  bout.reshape(1, D))
    return out


# speedup vs baseline: 2.3538x; 2.3538x over previous
import functools
import jax
import jax.numpy as jnp
from jax import lax
from jax.experimental import pallas as pl
from jax.experimental.pallas import tpu as pltpu
from jax.experimental.pallas import tpu_sc as plsc

N = 10000
D = 128
G = 64
E = 320000
NC = 2
NS = 16
NW = NC * NS
NP = 10240
RT = NP // NS
EP = 327680
CH = 80
C = 80
NB = 2
CPT = EP // NC // NS // C


@functools.cache
def _mesh():
    return plsc.VectorSubcoreMesh(
        core_axis_name="c", subcore_axis_name="s",
        num_cores=NC, num_subcores=NS)


def _deg_body(dst_hbm, ones_hbm, zeros_hbm, out_hbm, dst_vm, ones_vm, zero_vm,
              deg_sh):
    c = lax.axis_index("c")
    s = lax.axis_index("s")
    r0 = s * RT
    pltpu.sync_copy(zeros_hbm, zero_vm)
    for k in range(RT // 128):
        pltpu.sync_copy(zero_vm, deg_sh.at[pl.ds(r0 + k * 128, 128)])
    pltpu.sync_copy(ones_hbm, ones_vm)
    pltpu.sync_copy(dst_hbm.at[pl.ds((s * NC + c) * CH, CH)], dst_vm)
    plsc.subcore_barrier()

    def body(j, carry):
        pltpu.sync_copy(ones_vm, deg_sh.at[dst_vm.at[j]], add=True)
        return carry

    lax.fori_loop(0, CH, body, 0)
    plsc.subcore_barrier()
    pltpu.sync_copy(deg_sh.at[pl.ds(r0, RT)],
                    out_hbm.at[pl.ds(c * NP + r0, RT)])


@functools.cache
def _deg_kernel():
    return pl.kernel(
        _deg_body,
        out_type=jax.ShapeDtypeStruct((NC * NP, 8), jnp.float32),
        mesh=_mesh(),
        scratch_types=[
            pltpu.VMEM((CH, 128), jnp.int32),
            pltpu.VMEM((128, 8), jnp.float32),
            pltpu.VMEM((128, 8), jnp.float32),
            pltpu.VMEM_SHARED((NP, 8), jnp.float32),
        ],
        compiler_params=pltpu.CompilerParams(use_tc_tiling_on_sc=False),
    )


def _agg_body(src_hbm, dst_hbm, hs_hbm, zeros_hbm, out_hbm, src_vm, dst_vm,
              rbs, acc, sgs, sss):
    c = lax.axis_index("c")
    s = lax.axis_index("s")
    r0 = s * RT
    wid = c * NS + s
    pltpu.sync_copy(zeros_hbm, rbs[0])
    for k in range(RT // C):
        pltpu.sync_copy(rbs[0], acc.at[pl.ds(r0 + k * C, C)])
    pltpu.sync_copy(src_hbm.at[pl.ds(wid * CPT, CPT)], src_vm)
    pltpu.sync_copy(dst_hbm.at[pl.ds(wid * CPT, CPT)], dst_vm)
    plsc.subcore_barrier()

    for b in range(NB):
        pltpu.async_copy(hs_hbm.at[src_vm.at[b]], rbs[b], sgs[b])

    def body(i, carry):
        j = NB * i
        for b in range(NB):
            pltpu.make_async_copy(hs_hbm.at[src_vm.at[j + b]], rbs[b],
                                  sgs[b]).wait()
            pltpu.async_copy(rbs[b], acc.at[dst_vm.at[j + b]], sss[b],
                             add=True)
        for b in range(NB):
            @pl.when(j + NB + b < CPT)
            def _(b=b):
                pltpu.make_async_copy(rbs[b], acc.at[dst_vm.at[0]],
                                      sss[b]).wait()
                pltpu.async_copy(hs_hbm.at[src_vm.at[j + NB + b]], rbs[b],
                                 sgs[b])
        return carry

    lax.fori_loop(0, CPT // NB, body, 0)
    for b in range(NB):
        pltpu.make_async_copy(rbs[b], acc.at[dst_vm.at[0]], sss[b]).wait()
    plsc.subcore_barrier()
    pltpu.sync_copy(acc.at[pl.ds(r0, RT)],
                    out_hbm.at[pl.ds(c * NP + r0, RT)])


@functools.cache
def _agg_kernel():
    return pl.kernel(
        _agg_body,
        out_type=jax.ShapeDtypeStruct((NC * NP, D), jnp.float32),
        mesh=_mesh(),
        scratch_types=[
            pltpu.VMEM((CPT, C), jnp.int32),
            pltpu.VMEM((CPT, C), jnp.int32),
            [pltpu.VMEM((C, D), jnp.float32) for _ in range(NB)],
            pltpu.VMEM_SHARED((NP, D), jnp.float32),
            [pltpu.SemaphoreType.DMA for _ in range(NB)],
            [pltpu.SemaphoreType.DMA for _ in range(NB)],
        ],
        compiler_params=pltpu.CompilerParams(use_tc_tiling_on_sc=False),
    )


def _tc1_body(x_ref, degp_ref, w_ref, hs_ref, dinv_ref):
    deg = degp_ref[0] + degp_ref[1] + 1.0
    dinv = lax.rsqrt(deg)
    dinv_ref[...] = dinv
    hs_ref[...] = jnp.dot(x_ref[...] * dinv, w_ref[...],
                          preferred_element_type=jnp.float32)


def _tc_mid_body(acc_ref, hsp_ref, dinv_ref, b_ref, w_ref, hs_ref):
    dinv = dinv_ref[...]
    t = (acc_ref[0] + acc_ref[1] + hsp_ref[...]) * dinv + b_ref[...]
    t = jnp.maximum(t, 0.0)
    rid = lax.broadcasted_iota(jnp.int32, (NP, 1), 0)
    t = jnp.where(rid < N, t, 0.0)
    hs_ref[...] = jnp.dot(t * dinv, w_ref[...],
                          preferred_element_type=jnp.float32)


def _tc_fin_body(acc_ref, hsp_ref, dinv_ref, b_ref, batch_ref, wout_ref,
                 bout_ref, out_ref):
    h = (acc_ref[0] + acc_ref[1] + hsp_ref[...]) * dinv_ref[...] + b_ref[...]
    h = jnp.maximum(h, 0.0)
    oh = (batch_ref[...] == lax.broadcasted_iota(jnp.int32, (NP, G), 1)
          ).astype(jnp.float32)
    dn = (((0,), (0,)), ((), ()))
    sums = lax.dot_general(oh, h, dn, preferred_element_type=jnp.float32)
    cnt = lax.dot_general(oh, jnp.ones((NP, 1), jnp.float32), dn,
                          preferred_element_type=jnp.float32)
    pooled = sums / jnp.maximum(cnt, 1.0)
    out_ref[...] = (jnp.dot(pooled, wout_ref[...],
                            preferred_element_type=jnp.float32)
                    + bout_ref[...])


_tc1 = pl.pallas_call(
    _tc1_body,
    out_shape=[jax.ShapeDtypeStruct((NP, D), jnp.float32),
               jax.ShapeDtypeStruct((NP, 1), jnp.float32)],
)

_tc_mid = pl.pallas_call(
    _tc_mid_body,
    out_shape=jax.ShapeDtypeStruct((NP, D), jnp.float32),
)

_tc_fin = pl.pallas_call(
    _tc_fin_body,
    out_shape=jax.ShapeDtypeStruct((G, D), jnp.float32),
)


def kernel(x, edge_index, batch, W1, b1, W2, b2, W3, b3, Wout, bout):
    f32 = jnp.float32
    i32 = jnp.int32
    pad_e = N + (jnp.arange(EP - E, dtype=i32) % (NP - N))
    src = jnp.concatenate([edge_index[0].astype(i32), pad_e])
    dst = jnp.concatenate([edge_index[1].astype(i32), pad_e])
    src_w = src.reshape(NC * NS * CPT, C)
    dst_w = dst.reshape(NC * NS * CPT, C)
    dst_deg = dst.reshape(NW * CH, 128)
    x_pad = jnp.pad(x.astype(f32), ((0, NP - N), (0, 0)))
    batch_pad = jnp.pad(batch.astype(i32), (0, NP - N),
                        constant_values=G).reshape(NP, 1)
    zeros8 = jnp.zeros((128, 8), f32)
    ones8 = jnp.ones((128, 8), f32)
    zerosD = jnp.zeros((C, D), f32)

    deg_out = _deg_kernel()(dst_deg, ones8, zeros8)
    degp = deg_out.reshape(NC, NP, 8)[:, :, :1]

    agg = _agg_kernel()
    hs1, dinv = _tc1(x_pad, degp, W1)
    acc1 = agg(src_w, dst_w, hs1, zerosD).reshape(NC, NP, D)
    hs2 = _tc_mid(acc1, hs1, dinv, b1.reshape(1, D), W2)
    acc2 = agg(src_w, dst_w, hs2, zerosD).reshape(NC, NP, D)
    hs3 = _tc_mid(acc2, hs2, dinv, b2.reshape(1, D), W3)
    acc3 = agg(src_w, dst_w, hs3, zerosD).reshape(NC, NP, D)
    out = _tc_fin(acc3, hs3, dinv, b3.reshape(1, D), batch_pad, Wout,
                  bout.reshape(1, D))
    return out


# speedup vs baseline: 2.3616x; 1.0033x over previous
import functools
import jax
import jax.numpy as jnp
from jax import lax
from jax.experimental import pallas as pl
from jax.experimental.pallas import tpu as pltpu
from jax.experimental.pallas import tpu_sc as plsc

N = 10000
D = 128
G = 64
E = 320000
NC = 2
NS = 16
NW = NC * NS
NP = 10240
RT = NP // NS
EP = 327680
CH = 80
C = 80
NB = 2
CPT = EP // NC // NS // C


@functools.cache
def _mesh():
    return plsc.VectorSubcoreMesh(
        core_axis_name="c", subcore_axis_name="s",
        num_cores=NC, num_subcores=NS)


def _deg_body(dst_hbm, ones_hbm, zeros_hbm, out_hbm, dst_vm, ones_vm, zero_vm,
              deg_sh):
    c = lax.axis_index("c")
    s = lax.axis_index("s")
    r0 = s * RT
    pltpu.sync_copy(zeros_hbm, zero_vm)
    for k in range(RT // 128):
        pltpu.sync_copy(zero_vm, deg_sh.at[pl.ds(r0 + k * 128, 128)])
    pltpu.sync_copy(ones_hbm, ones_vm)
    pltpu.sync_copy(dst_hbm.at[pl.ds((s * NC + c) * CH, CH)], dst_vm)
    plsc.subcore_barrier()

    def body(j, carry):
        pltpu.sync_copy(ones_vm, deg_sh.at[dst_vm.at[j]], add=True)
        return carry

    lax.fori_loop(0, CH, body, 0)
    plsc.subcore_barrier()
    pltpu.sync_copy(deg_sh.at[pl.ds(r0, RT)],
                    out_hbm.at[pl.ds(c * NP + r0, RT)])


@functools.cache
def _deg_kernel():
    return pl.kernel(
        _deg_body,
        out_type=jax.ShapeDtypeStruct((NC * NP, 8), jnp.float32),
        mesh=_mesh(),
        scratch_types=[
            pltpu.VMEM((CH, 128), jnp.int32),
            pltpu.VMEM((128, 8), jnp.float32),
            pltpu.VMEM((128, 8), jnp.float32),
            pltpu.VMEM_SHARED((NP, 8), jnp.float32),
        ],
        compiler_params=pltpu.CompilerParams(use_tc_tiling_on_sc=False),
    )


def _agg_body(src_hbm, dst_hbm, hs_hbm, zeros_hbm, out_hbm, src_vm, dst_vm,
              rbs, acc, sgs, sss):
    c = lax.axis_index("c")
    s = lax.axis_index("s")
    r0 = s * RT
    wid = c * NS + s
    pltpu.sync_copy(src_hbm.at[pl.ds(wid * CPT, CPT)], src_vm)
    pltpu.sync_copy(dst_hbm.at[pl.ds(wid * CPT, CPT)], dst_vm)
    for b in range(1, NB):
        pltpu.async_copy(hs_hbm.at[src_vm.at[b]], rbs[b], sgs[b])
    pltpu.sync_copy(zeros_hbm, rbs[0])
    for k in range(RT // C):
        pltpu.sync_copy(rbs[0], acc.at[pl.ds(r0 + k * C, C)])
    pltpu.async_copy(hs_hbm.at[src_vm.at[0]], rbs[0], sgs[0])
    plsc.subcore_barrier()


    def body(i, carry):
        j = NB * i
        for b in range(NB):
            pltpu.make_async_copy(hs_hbm.at[src_vm.at[j + b]], rbs[b],
                                  sgs[b]).wait()
            pltpu.async_copy(rbs[b], acc.at[dst_vm.at[j + b]], sss[b],
                             add=True)
        for b in range(NB):
            @pl.when(j + NB + b < CPT)
            def _(b=b):
                pltpu.make_async_copy(rbs[b], acc.at[dst_vm.at[0]],
                                      sss[b]).wait()
                pltpu.async_copy(hs_hbm.at[src_vm.at[j + NB + b]], rbs[b],
                                 sgs[b])
        return carry

    lax.fori_loop(0, CPT // NB, body, 0)
    for b in range(NB):
        pltpu.make_async_copy(rbs[b], acc.at[dst_vm.at[0]], sss[b]).wait()
    plsc.subcore_barrier()
    pltpu.sync_copy(acc.at[pl.ds(r0, RT)],
                    out_hbm.at[pl.ds(c * NP + r0, RT)])


@functools.cache
def _agg_kernel():
    return pl.kernel(
        _agg_body,
        out_type=jax.ShapeDtypeStruct((NC * NP, D), jnp.float32),
        mesh=_mesh(),
        scratch_types=[
            pltpu.VMEM((CPT, C), jnp.int32),
            pltpu.VMEM((CPT, C), jnp.int32),
            [pltpu.VMEM((C, D), jnp.float32) for _ in range(NB)],
            pltpu.VMEM_SHARED((NP, D), jnp.float32),
            [pltpu.SemaphoreType.DMA for _ in range(NB)],
            [pltpu.SemaphoreType.DMA for _ in range(NB)],
        ],
        compiler_params=pltpu.CompilerParams(use_tc_tiling_on_sc=False),
    )


def _tc1_body(x_ref, degp_ref, w_ref, hs_ref, dinv_ref):
    deg = degp_ref[0] + degp_ref[1] + 1.0
    dinv = lax.rsqrt(deg)
    dinv_ref[...] = dinv
    hs_ref[...] = jnp.dot(x_ref[...] * dinv, w_ref[...],
                          preferred_element_type=jnp.float32)


def _tc_mid_body(acc_ref, hsp_ref, dinv_ref, b_ref, w_ref, hs_ref):
    dinv = dinv_ref[...]
    t = (acc_ref[0] + acc_ref[1] + hsp_ref[...]) * dinv + b_ref[...]
    t = jnp.maximum(t, 0.0)
    rid = lax.broadcasted_iota(jnp.int32, (NP, 1), 0)
    t = jnp.where(rid < N, t, 0.0)
    hs_ref[...] = jnp.dot(t * dinv, w_ref[...],
                          preferred_element_type=jnp.float32)


def _tc_fin_body(acc_ref, hsp_ref, dinv_ref, b_ref, batch_ref, wout_ref,
                 bout_ref, out_ref):
    h = (acc_ref[0] + acc_ref[1] + hsp_ref[...]) * dinv_ref[...] + b_ref[...]
    h = jnp.maximum(h, 0.0)
    oh = (batch_ref[...] == lax.broadcasted_iota(jnp.int32, (NP, G), 1)
          ).astype(jnp.float32)
    dn = (((0,), (0,)), ((), ()))
    sums = lax.dot_general(oh, h, dn, preferred_element_type=jnp.float32)
    cnt = lax.dot_general(oh, jnp.ones((NP, 1), jnp.float32), dn,
                          preferred_element_type=jnp.float32)
    pooled = sums / jnp.maximum(cnt, 1.0)
    out_ref[...] = (jnp.dot(pooled, wout_ref[...],
                            preferred_element_type=jnp.float32)
                    + bout_ref[...])


_tc1 = pl.pallas_call(
    _tc1_body,
    out_shape=[jax.ShapeDtypeStruct((NP, D), jnp.float32),
               jax.ShapeDtypeStruct((NP, 1), jnp.float32)],
)

_tc_mid = pl.pallas_call(
    _tc_mid_body,
    out_shape=jax.ShapeDtypeStruct((NP, D), jnp.float32),
)

_tc_fin = pl.pallas_call(
    _tc_fin_body,
    out_shape=jax.ShapeDtypeStruct((G, D), jnp.float32),
)


def kernel(x, edge_index, batch, W1, b1, W2, b2, W3, b3, Wout, bout):
    f32 = jnp.float32
    i32 = jnp.int32
    pad_e = N + (jnp.arange(EP - E, dtype=i32) % (NP - N))
    src = jnp.concatenate([edge_index[0].astype(i32), pad_e])
    dst = jnp.concatenate([edge_index[1].astype(i32), pad_e])
    src_w = src.reshape(NC * NS * CPT, C)
    dst_w = dst.reshape(NC * NS * CPT, C)
    dst_deg = dst.reshape(NW * CH, 128)
    x_pad = jnp.pad(x.astype(f32), ((0, NP - N), (0, 0)))
    batch_pad = jnp.pad(batch.astype(i32), (0, NP - N),
                        constant_values=G).reshape(NP, 1)
    zeros8 = jnp.zeros((128, 8), f32)
    ones8 = jnp.ones((128, 8), f32)
    zerosD = jnp.zeros((C, D), f32)

    deg_out = _deg_kernel()(dst_deg, ones8, zeros8)
    degp = deg_out.reshape(NC, NP, 8)[:, :, :1]

    agg = _agg_kernel()
    hs1, dinv = _tc1(x_pad, degp, W1)
    acc1 = agg(src_w, dst_w, hs1, zerosD).reshape(NC, NP, D)
    hs2 = _tc_mid(acc1, hs1, dinv, b1.reshape(1, D), W2)
    acc2 = agg(src_w, dst_w, hs2, zerosD).reshape(NC, NP, D)
    hs3 = _tc_mid(acc2, hs2, dinv, b2.reshape(1, D), W3)
    acc3 = agg(src_w, dst_w, hs3, zerosD).reshape(NC, NP, D)
    out = _tc_fin(acc3, hs3, dinv, b3.reshape(1, D), batch_pad, Wout,
                  bout.reshape(1, D))
    return out


# speedup vs baseline: 2.9130x; 1.2335x over previous
import functools
import jax
import jax.numpy as jnp
from jax import lax
from jax.experimental import pallas as pl
from jax.experimental.pallas import tpu as pltpu
from jax.experimental.pallas import tpu_sc as plsc

N = 10000
D = 128
G = 64
E = 320000
NC = 2
NS = 16
NW = NC * NS
NP = 10240
RT = NP // NS
EP = 327680
CH = 80
C = 40
NB = 4
CPT = EP // NC // NS // C


@functools.cache
def _mesh():
    return plsc.VectorSubcoreMesh(
        core_axis_name="c", subcore_axis_name="s",
        num_cores=NC, num_subcores=NS)


def _deg_body(dst_hbm, ones_hbm, zeros_hbm, out_hbm, dst_vm, ones_vm, zero_vm,
              deg_sh):
    c = lax.axis_index("c")
    s = lax.axis_index("s")
    r0 = s * RT
    pltpu.sync_copy(zeros_hbm, zero_vm)
    for k in range(RT // 128):
        pltpu.sync_copy(zero_vm, deg_sh.at[pl.ds(r0 + k * 128, 128)])
    pltpu.sync_copy(ones_hbm, ones_vm)
    pltpu.sync_copy(dst_hbm.at[pl.ds((s * NC + c) * CH, CH)], dst_vm)
    plsc.subcore_barrier()

    def body(j, carry):
        pltpu.sync_copy(ones_vm, deg_sh.at[dst_vm.at[j]], add=True)
        return carry

    lax.fori_loop(0, CH, body, 0)
    plsc.subcore_barrier()
    pltpu.sync_copy(deg_sh.at[pl.ds(r0, RT)],
                    out_hbm.at[pl.ds(c * NP + r0, RT)])


@functools.cache
def _deg_kernel():
    return pl.kernel(
        _deg_body,
        out_type=jax.ShapeDtypeStruct((NC * NP, 8), jnp.float32),
        mesh=_mesh(),
        scratch_types=[
            pltpu.VMEM((CH, 128), jnp.int32),
            pltpu.VMEM((128, 8), jnp.float32),
            pltpu.VMEM((128, 8), jnp.float32),
            pltpu.VMEM_SHARED((NP, 8), jnp.float32),
        ],
        compiler_params=pltpu.CompilerParams(use_tc_tiling_on_sc=False),
    )


def _agg_body(src_hbm, dst_hbm, hs_hbm, zeros_hbm, out_hbm, src_vm, dst_vm,
              rbs, acc, sgs, sss):
    c = lax.axis_index("c")
    s = lax.axis_index("s")
    r0 = s * RT
    wid = c * NS + s
    pltpu.sync_copy(src_hbm.at[pl.ds(wid * CPT, CPT)], src_vm)
    pltpu.sync_copy(dst_hbm.at[pl.ds(wid * CPT, CPT)], dst_vm)
    for b in range(1, NB):
        pltpu.async_copy(hs_hbm.at[src_vm.at[b]], rbs[b], sgs[b])
    pltpu.sync_copy(zeros_hbm, rbs[0])
    for k in range(RT // C):
        pltpu.sync_copy(rbs[0], acc.at[pl.ds(r0 + k * C, C)])
    pltpu.async_copy(hs_hbm.at[src_vm.at[0]], rbs[0], sgs[0])
    plsc.subcore_barrier()


    def body(i, carry):
        j = NB * i
        for b in range(NB):
            pltpu.make_async_copy(hs_hbm.at[src_vm.at[j + b]], rbs[b],
                                  sgs[b]).wait()
            pltpu.async_copy(rbs[b], acc.at[dst_vm.at[j + b]], sss[b],
                             add=True)
        for b in range(NB):
            @pl.when(j + NB + b < CPT)
            def _(b=b):
                pltpu.make_async_copy(rbs[b], acc.at[dst_vm.at[0]],
                                      sss[b]).wait()
                pltpu.async_copy(hs_hbm.at[src_vm.at[j + NB + b]], rbs[b],
                                 sgs[b])
        return carry

    lax.fori_loop(0, CPT // NB, body, 0)
    for b in range(NB):
        pltpu.make_async_copy(rbs[b], acc.at[dst_vm.at[0]], sss[b]).wait()
    plsc.subcore_barrier()
    pltpu.sync_copy(acc.at[pl.ds(r0, RT)],
                    out_hbm.at[pl.ds(c * NP + r0, RT)])


@functools.cache
def _agg_kernel():
    return pl.kernel(
        _agg_body,
        out_type=jax.ShapeDtypeStruct((NC * NP, D), jnp.float32),
        mesh=_mesh(),
        scratch_types=[
            pltpu.VMEM((CPT, C), jnp.int32),
            pltpu.VMEM((CPT, C), jnp.int32),
            [pltpu.VMEM((C, D), jnp.float32) for _ in range(NB)],
            pltpu.VMEM_SHARED((NP, D), jnp.float32),
            [pltpu.SemaphoreType.DMA for _ in range(NB)],
            [pltpu.SemaphoreType.DMA for _ in range(NB)],
        ],
        compiler_params=pltpu.CompilerParams(use_tc_tiling_on_sc=False),
    )


def _tc1_body(x_ref, degp_ref, w_ref, hs_ref, dinv_ref):
    deg = degp_ref[0] + degp_ref[1] + 1.0
    dinv = lax.rsqrt(deg)
    dinv_ref[...] = dinv
    hs_ref[...] = jnp.dot(x_ref[...] * dinv, w_ref[...],
                          preferred_element_type=jnp.float32)


def _tc_mid_body(acc_ref, hsp_ref, dinv_ref, b_ref, w_ref, hs_ref):
    dinv = dinv_ref[...]
    t = (acc_ref[0] + acc_ref[1] + hsp_ref[...]) * dinv + b_ref[...]
    t = jnp.maximum(t, 0.0)
    rid = lax.broadcasted_iota(jnp.int32, (NP, 1), 0)
    t = jnp.where(rid < N, t, 0.0)
    hs_ref[...] = jnp.dot(t * dinv, w_ref[...],
                          preferred_element_type=jnp.float32)


def _tc_fin_body(acc_ref, hsp_ref, dinv_ref, b_ref, batch_ref, wout_ref,
                 bout_ref, out_ref):
    h = (acc_ref[0] + acc_ref[1] + hsp_ref[...]) * dinv_ref[...] + b_ref[...]
    h = jnp.maximum(h, 0.0)
    oh = (batch_ref[...] == lax.broadcasted_iota(jnp.int32, (NP, G), 1)
          ).astype(jnp.float32)
    dn = (((0,), (0,)), ((), ()))
    sums = lax.dot_general(oh, h, dn, preferred_element_type=jnp.float32)
    cnt = lax.dot_general(oh, jnp.ones((NP, 1), jnp.float32), dn,
                          preferred_element_type=jnp.float32)
    pooled = sums / jnp.maximum(cnt, 1.0)
    out_ref[...] = (jnp.dot(pooled, wout_ref[...],
                            preferred_element_type=jnp.float32)
                    + bout_ref[...])


_tc1 = pl.pallas_call(
    _tc1_body,
    out_shape=[jax.ShapeDtypeStruct((NP, D), jnp.float32),
               jax.ShapeDtypeStruct((NP, 1), jnp.float32)],
)

_tc_mid = pl.pallas_call(
    _tc_mid_body,
    out_shape=jax.ShapeDtypeStruct((NP, D), jnp.float32),
)

_tc_fin = pl.pallas_call(
    _tc_fin_body,
    out_shape=jax.ShapeDtypeStruct((G, D), jnp.float32),
)


def kernel(x, edge_index, batch, W1, b1, W2, b2, W3, b3, Wout, bout):
    f32 = jnp.float32
    i32 = jnp.int32
    pad_e = N + (jnp.arange(EP - E, dtype=i32) % (NP - N))
    src = jnp.concatenate([edge_index[0].astype(i32), pad_e])
    dst = jnp.concatenate([edge_index[1].astype(i32), pad_e])
    src_w = src.reshape(NC * NS * CPT, C)
    dst_w = dst.reshape(NC * NS * CPT, C)
    dst_deg = dst.reshape(NW * CH, 128)
    x_pad = jnp.pad(x.astype(f32), ((0, NP - N), (0, 0)))
    batch_pad = jnp.pad(batch.astype(i32), (0, NP - N),
                        constant_values=G).reshape(NP, 1)
    zeros8 = jnp.zeros((128, 8), f32)
    ones8 = jnp.ones((128, 8), f32)
    zerosD = jnp.zeros((C, D), f32)

    deg_out = _deg_kernel()(dst_deg, ones8, zeros8)
    degp = deg_out.reshape(NC, NP, 8)[:, :, :1]

    agg = _agg_kernel()
    hs1, dinv = _tc1(x_pad, degp, W1)
    acc1 = agg(src_w, dst_w, hs1, zerosD).reshape(NC, NP, D)
    hs2 = _tc_mid(acc1, hs1, dinv, b1.reshape(1, D), W2)
    acc2 = agg(src_w, dst_w, hs2, zerosD).reshape(NC, NP, D)
    hs3 = _tc_mid(acc2, hs2, dinv, b2.reshape(1, D), W3)
    acc3 = agg(src_w, dst_w, hs3, zerosD).reshape(NC, NP, D)
    out = _tc_fin(acc3, hs3, dinv, b3.reshape(1, D), batch_pad, Wout,
                  bout.reshape(1, D))
    return out


# speedup vs baseline: 2.9276x; 1.0050x over previous
import functools
import jax
import jax.numpy as jnp
from jax import lax
from jax.experimental import pallas as pl
from jax.experimental.pallas import tpu as pltpu
from jax.experimental.pallas import tpu_sc as plsc

N = 10000
D = 128
G = 64
E = 320000
NC = 2
NS = 16
NW = NC * NS
NP = 10240
RT = NP // NS
EP = 327680
CH = 80
C = 32
NB = 5
CPT = EP // NC // NS // C


@functools.cache
def _mesh():
    return plsc.VectorSubcoreMesh(
        core_axis_name="c", subcore_axis_name="s",
        num_cores=NC, num_subcores=NS)


def _deg_body(dst_hbm, ones_hbm, zeros_hbm, out_hbm, dst_vm, ones_vm, zero_vm,
              deg_sh):
    c = lax.axis_index("c")
    s = lax.axis_index("s")
    r0 = s * RT
    pltpu.sync_copy(zeros_hbm, zero_vm)
    for k in range(RT // 128):
        pltpu.sync_copy(zero_vm, deg_sh.at[pl.ds(r0 + k * 128, 128)])
    pltpu.sync_copy(ones_hbm, ones_vm)
    pltpu.sync_copy(dst_hbm.at[pl.ds((s * NC + c) * CH, CH)], dst_vm)
    plsc.subcore_barrier()

    def body(j, carry):
        pltpu.sync_copy(ones_vm, deg_sh.at[dst_vm.at[j]], add=True)
        return carry

    lax.fori_loop(0, CH, body, 0)
    plsc.subcore_barrier()
    pltpu.sync_copy(deg_sh.at[pl.ds(r0, RT)],
                    out_hbm.at[pl.ds(c * NP + r0, RT)])


@functools.cache
def _deg_kernel():
    return pl.kernel(
        _deg_body,
        out_type=jax.ShapeDtypeStruct((NC * NP, 8), jnp.float32),
        mesh=_mesh(),
        scratch_types=[
            pltpu.VMEM((CH, 128), jnp.int32),
            pltpu.VMEM((128, 8), jnp.float32),
            pltpu.VMEM((128, 8), jnp.float32),
            pltpu.VMEM_SHARED((NP, 8), jnp.float32),
        ],
        compiler_params=pltpu.CompilerParams(use_tc_tiling_on_sc=False),
    )


def _agg_body(src_hbm, dst_hbm, hs_hbm, zeros_hbm, out_hbm, src_vm, dst_vm,
              rbs, acc, sgs, sss):
    c = lax.axis_index("c")
    s = lax.axis_index("s")
    r0 = s * RT
    wid = c * NS + s
    pltpu.sync_copy(src_hbm.at[pl.ds(wid * CPT, CPT)], src_vm)
    pltpu.sync_copy(dst_hbm.at[pl.ds(wid * CPT, CPT)], dst_vm)
    for b in range(1, NB):
        pltpu.async_copy(hs_hbm.at[src_vm.at[b]], rbs[b], sgs[b])
    pltpu.sync_copy(zeros_hbm, rbs[0])
    for k in range(RT // C):
        pltpu.sync_copy(rbs[0], acc.at[pl.ds(r0 + k * C, C)])
    pltpu.async_copy(hs_hbm.at[src_vm.at[0]], rbs[0], sgs[0])
    plsc.subcore_barrier()


    def body(i, carry):
        j = NB * i
        for b in range(NB):
            pltpu.make_async_copy(hs_hbm.at[src_vm.at[j + b]], rbs[b],
                                  sgs[b]).wait()
            pltpu.async_copy(rbs[b], acc.at[dst_vm.at[j + b]], sss[b],
                             add=True)
        for b in range(NB):
            @pl.when(j + NB + b < CPT)
            def _(b=b):
                pltpu.make_async_copy(rbs[b], acc.at[dst_vm.at[0]],
                                      sss[b]).wait()
                pltpu.async_copy(hs_hbm.at[src_vm.at[j + NB + b]], rbs[b],
                                 sgs[b])
        return carry

    lax.fori_loop(0, CPT // NB, body, 0)
    for b in range(NB):
        pltpu.make_async_copy(rbs[b], acc.at[dst_vm.at[0]], sss[b]).wait()
    plsc.subcore_barrier()
    pltpu.sync_copy(acc.at[pl.ds(r0, RT)],
                    out_hbm.at[pl.ds(c * NP + r0, RT)])


@functools.cache
def _agg_kernel():
    return pl.kernel(
        _agg_body,
        out_type=jax.ShapeDtypeStruct((NC * NP, D), jnp.float32),
        mesh=_mesh(),
        scratch_types=[
            pltpu.VMEM((CPT, C), jnp.int32),
            pltpu.VMEM((CPT, C), jnp.int32),
            [pltpu.VMEM((C, D), jnp.float32) for _ in range(NB)],
            pltpu.VMEM_SHARED((NP, D), jnp.float32),
            [pltpu.SemaphoreType.DMA for _ in range(NB)],
            [pltpu.SemaphoreType.DMA for _ in range(NB)],
        ],
        compiler_params=pltpu.CompilerParams(use_tc_tiling_on_sc=False),
    )


def _tc1_body(x_ref, degp_ref, w_ref, hs_ref, dinv_ref):
    deg = degp_ref[0] + degp_ref[1] + 1.0
    dinv = lax.rsqrt(deg)
    dinv_ref[...] = dinv
    hs_ref[...] = jnp.dot(x_ref[...] * dinv, w_ref[...],
                          preferred_element_type=jnp.float32)


def _tc_mid_body(acc_ref, hsp_ref, dinv_ref, b_ref, w_ref, hs_ref):
    dinv = dinv_ref[...]
    t = (acc_ref[0] + acc_ref[1] + hsp_ref[...]) * dinv + b_ref[...]
    t = jnp.maximum(t, 0.0)
    rid = lax.broadcasted_iota(jnp.int32, (NP, 1), 0)
    t = jnp.where(rid < N, t, 0.0)
    hs_ref[...] = jnp.dot(t * dinv, w_ref[...],
                          preferred_element_type=jnp.float32)


def _tc_fin_body(acc_ref, hsp_ref, dinv_ref, b_ref, batch_ref, wout_ref,
                 bout_ref, out_ref):
    h = (acc_ref[0] + acc_ref[1] + hsp_ref[...]) * dinv_ref[...] + b_ref[...]
    h = jnp.maximum(h, 0.0)
    oh = (batch_ref[...] == lax.broadcasted_iota(jnp.int32, (NP, G), 1)
          ).astype(jnp.float32)
    dn = (((0,), (0,)), ((), ()))
    sums = lax.dot_general(oh, h, dn, preferred_element_type=jnp.float32)
    cnt = lax.dot_general(oh, jnp.ones((NP, 1), jnp.float32), dn,
                          preferred_element_type=jnp.float32)
    pooled = sums / jnp.maximum(cnt, 1.0)
    out_ref[...] = (jnp.dot(pooled, wout_ref[...],
                            preferred_element_type=jnp.float32)
                    + bout_ref[...])


_tc1 = pl.pallas_call(
    _tc1_body,
    out_shape=[jax.ShapeDtypeStruct((NP, D), jnp.float32),
               jax.ShapeDtypeStruct((NP, 1), jnp.float32)],
)

_tc_mid = pl.pallas_call(
    _tc_mid_body,
    out_shape=jax.ShapeDtypeStruct((NP, D), jnp.float32),
)

_tc_fin = pl.pallas_call(
    _tc_fin_body,
    out_shape=jax.ShapeDtypeStruct((G, D), jnp.float32),
)


def kernel(x, edge_index, batch, W1, b1, W2, b2, W3, b3, Wout, bout):
    f32 = jnp.float32
    i32 = jnp.int32
    pad_e = N + (jnp.arange(EP - E, dtype=i32) % (NP - N))
    src = jnp.concatenate([edge_index[0].astype(i32), pad_e])
    dst = jnp.concatenate([edge_index[1].astype(i32), pad_e])
    src_w = src.reshape(NC * NS * CPT, C)
    dst_w = dst.reshape(NC * NS * CPT, C)
    dst_deg = dst.reshape(NW * CH, 128)
    x_pad = jnp.pad(x.astype(f32), ((0, NP - N), (0, 0)))
    batch_pad = jnp.pad(batch.astype(i32), (0, NP - N),
                        constant_values=G).reshape(NP, 1)
    zeros8 = jnp.zeros((128, 8), f32)
    ones8 = jnp.ones((128, 8), f32)
    zerosD = jnp.zeros((C, D), f32)

    deg_out = _deg_kernel()(dst_deg, ones8, zeros8)
    degp = deg_out.reshape(NC, NP, 8)[:, :, :1]

    agg = _agg_kernel()
    hs1, dinv = _tc1(x_pad, degp, W1)
    acc1 = agg(src_w, dst_w, hs1, zerosD).reshape(NC, NP, D)
    hs2 = _tc_mid(acc1, hs1, dinv, b1.reshape(1, D), W2)
    acc2 = agg(src_w, dst_w, hs2, zerosD).reshape(NC, NP, D)
    hs3 = _tc_mid(acc2, hs2, dinv, b2.reshape(1, D), W3)
    acc3 = agg(src_w, dst_w, hs3, zerosD).reshape(NC, NP, D)
    out = _tc_fin(acc3, hs3, dinv, b3.reshape(1, D), batch_pad, Wout,
                  bout.reshape(1, D))
    return out
